# baseline (speedup 1.0000x reference)
import functools

import jax
import jax.numpy as jnp
from jax import lax
from jax.experimental import pallas as pl
from jax.experimental.pallas import tpu as pltpu
from jax.experimental.pallas import tpu_sc as plsc

_V = 1000000
_ROWS = 16384
_SEQ = 200
_SEQP = 256
_D = 64
_NW = 32
_RPW = _ROWS // _NW
_R = 2
_C = _R * _SEQ
_C2 = _C // 2
_C2P = 208
_NCHUNK = _RPW // _R
_NSTAGE = _NCHUNK // 4
_OUT_ROWS = _ROWS * _SEQ * _D // 128

_mesh = plsc.VectorSubcoreMesh(core_axis_name="c", subcore_axis_name="s")


@functools.partial(
    pl.kernel,
    mesh=_mesh,
    out_type=jax.ShapeDtypeStruct((_OUT_ROWS, 128), jnp.float32),
    compiler_params=pltpu.CompilerParams(needs_layout_passes=False),
    scratch_types=[
        pltpu.VMEM((2048,), jnp.int32),
        pltpu.VMEM((2048,), jnp.int32),
        pltpu.VMEM((_C2P,), jnp.int32),
        pltpu.VMEM((_C2P,), jnp.int32),
        pltpu.VMEM((_C2P,), jnp.int32),
        pltpu.VMEM((_C2P,), jnp.int32),
        pltpu.VMEM((_C2P, 128), jnp.float32),
        pltpu.VMEM((_C2P, 128), jnp.float32),
        pltpu.VMEM((_C2P, 128), jnp.float32),
        pltpu.VMEM((_C2P, 128), jnp.float32),
        pltpu.SemaphoreType.DMA,
        pltpu.SemaphoreType.DMA,
        pltpu.SemaphoreType.DMA,
        pltpu.SemaphoreType.DMA,
        pltpu.SemaphoreType.DMA,
        pltpu.SemaphoreType.DMA,
    ],
)
def _embed_sc(idx_hbm, table_hbm, out_hbm,
              idx_v0, idx_v1, ie0, ie1, io0, io1,
              re0, re1, ro0, ro1,
              si0, si1, sg0, sg1, so0, so1):
    wid = lax.axis_index("s") * 2 + lax.axis_index("c")
    base = wid * _RPW
    pad_row = wid * 997

    idx_v = (idx_v0, idx_v1)
    idx_e = (ie0, ie1)
    idx_o = (io0, io1)
    rows_e = (re0, re1)
    rows_o = (ro0, ro1)
    sem_i = (si0, si1)
    sem_g = (sg0, sg1)
    sem_o = (so0, so1)

    def idx_start(stage, b):
        w0 = (base + stage * 8) * _SEQP
        pltpu.make_async_copy(
            idx_hbm.at[pl.ds(w0, 2048)], idx_v[b], sem_i[b]).start()

    def idx_wait(b):
        pltpu.make_async_copy(
            idx_hbm.at[pl.ds(0, 2048)], idx_v[b], sem_i[b]).wait()

    def deinterleave(src_b, q, b):
        lane = lax.iota(jnp.int32, 16)
        q512 = q * (2 * _SEQP)
        for k in range(_C2P // 16):
            e = lane + (16 * k)
            pos = q512 + (e // 100) * _SEQP + (e % 100) * 2
            ev = plsc.load_gather(idx_v[src_b], [pos])
            od = plsc.load_gather(idx_v[src_b], [pos + 1])
            if (k + 1) * 16 > _C2:
                ok = e < _C2
                ev = jnp.where(ok, ev, pad_row)
                od = jnp.where(ok, od, pad_row)
            idx_e[b][pl.ds(16 * k, 16)] = ev
            idx_o[b][pl.ds(16 * k, 16)] = od

    def gather_start(b):
        pltpu.make_async_copy(
            table_hbm.at[idx_e[b]], rows_e[b], sem_g[b]).start()
        pltpu.make_async_copy(
            table_hbm.at[idx_o[b]], rows_o[b], sem_g[b]).start()

    def gather_wait(b):
        pltpu.make_async_copy(
            table_hbm.at[idx_e[b]], rows_e[b], sem_g[b]).wait()
        pltpu.make_async_copy(
            table_hbm.at[idx_o[b]], rows_o[b], sem_g[b]).wait()

    def merge(b):
        @pl.loop(0, _C2, unroll=8)
        def _row(k):
            for j in range(_D // 16):
                rows_e[b][k, pl.ds(_D + 16 * j, 16)] = (
                    rows_o[b][k, pl.ds(16 * j, 16)])

    def scatter_start(i, b):
        r0 = (base + i * _R) * (_SEQ * _D // 128)
        pltpu.make_async_copy(
            rows_e[b].at[pl.ds(0, _C2), :],
            out_hbm.at[pl.ds(r0, _C2), :], sem_o[b]).start()

    def scatter_wait(b):
        pltpu.make_async_copy(
            rows_e[b].at[pl.ds(0, _C2), :],
            out_hbm.at[pl.ds(0, _C2), :], sem_o[b]).wait()

    idx_start(0, 0)
    idx_wait(0)
    deinterleave(0, 0, 0)
    idx_start(1, 1)
    gather_start(0)

    @pl.loop(0, _NCHUNK, step=2)
    def _pair(i):
        for b in (0, 1):
            chunk = i + b
            nb = 1 - b
            nxt = chunk + 1

            @pl.when(nxt < _NCHUNK)
            def _():
                m8 = lax.rem(nxt, 8)
                @pl.when(m8 == 0)
                def _():
                    idx_wait(0)
                    @pl.when(nxt + 4 < _NCHUNK)
                    def _():
                        idx_start(nxt // 4 + 1, 1)
                @pl.when(m8 == 4)
                def _():
                    idx_wait(1)
                    @pl.when(nxt + 4 < _NCHUNK)
                    def _():
                        idx_start(nxt // 4 + 1, 0)
                q = lax.rem(nxt, 4)
                @pl.when(m8 < 4)
                def _():
                    deinterleave(0, q, nb)
                @pl.when(m8 >= 4)
                def _():
                    deinterleave(1, q, nb)
                @pl.when(chunk >= 1)
                def _():
                    scatter_wait(nb)
                gather_start(nb)

            gather_wait(b)
            merge(b)
            scatter_start(chunk, b)

    scatter_wait(0)
    scatter_wait(1)


def kernel(input, weight):
    idx = jnp.pad(jnp.asarray(input, jnp.int32),
                  ((0, 0), (0, _SEQP - _SEQ))).reshape(_ROWS * _SEQP)
    table = jnp.pad(weight, ((0, 0), (0, 128 - _D)))
    out = _embed_sc(idx, table)
    return out.reshape(_ROWS, _SEQ, _D)


# speedup vs baseline: 1.9307x; 1.9307x over previous
import functools

import jax
import jax.numpy as jnp
from jax import lax
from jax.experimental import pallas as pl
from jax.experimental.pallas import tpu as pltpu
from jax.experimental.pallas import tpu_sc as plsc

_ROWS = 16384
_SEQ = 200
_D = 64
_B = _ROWS * _SEQ
_NW = 32
_BPW = _B // _NW
_C = 800
_NCHUNK = _BPW // _C

_mesh = plsc.VectorSubcoreMesh(core_axis_name="c", subcore_axis_name="s")


@functools.partial(
    pl.kernel,
    mesh=_mesh,
    out_type=jax.ShapeDtypeStruct((_B, 128), jnp.float32),
    scratch_types=[
        pltpu.VMEM((_C,), jnp.int32),
        pltpu.VMEM((_C,), jnp.int32),
        pltpu.VMEM((_C, _D), jnp.float32),
        pltpu.VMEM((_C, _D), jnp.float32),
        pltpu.SemaphoreType.DMA,
        pltpu.SemaphoreType.DMA,
        pltpu.SemaphoreType.DMA,
        pltpu.SemaphoreType.DMA,
        pltpu.SemaphoreType.DMA,
        pltpu.SemaphoreType.DMA,
    ],
    compiler_params=pltpu.CompilerParams(
        use_tc_tiling_on_sc=False, needs_layout_passes=False),
)
def _embed_sc(idx_hbm, table_hbm, out_hbm, idx_v0, idx_v1, rows_v0, rows_v1,
              si0, si1, sg0, sg1, so0, so1):
    wid = lax.axis_index("s") * 2 + lax.axis_index("c")
    base = wid * _BPW

    idx_v = (idx_v0, idx_v1)
    rows_v = (rows_v0, rows_v1)
    sem_i = (si0, si1)
    sem_g = (sg0, sg1)
    sem_o = (so0, so1)

    def idx_desc(i, b):
        return pltpu.make_async_copy(
            idx_hbm.at[pl.ds(base + i * _C, _C)], idx_v[b], sem_i[b])

    def gather_desc(b):
        return pltpu.make_async_copy(
            table_hbm.at[idx_v[b]], rows_v[b], sem_g[b])

    def scatter_desc(i, b):
        return pltpu.make_async_copy(
            rows_v[b],
            out_hbm.at[pl.ds(base + i * _C, _C), pl.ds(0, _D)], sem_o[b])

    d = idx_desc(0, 0)
    d.start()
    d.wait()
    idx_desc(1, 1).start()
    gather_desc(0).start()

    @pl.loop(0, _NCHUNK, step=2)
    def _pair(i):
        for b in (0, 1):
            chunk = i + b
            nb = 1 - b

            @pl.when(chunk + 1 < _NCHUNK)
            def _():
                idx_desc(0, nb).wait()
                @pl.when(chunk >= 1)
                def _():
                    scatter_desc(0, nb).wait()
                gather_desc(nb).start()

            gather_desc(b).wait()
            @pl.when(chunk + 2 < _NCHUNK)
            def _():
                idx_desc(chunk + 2, b).start()
            scatter_desc(chunk, b).start()

    scatter_desc(0, 0).wait()
    scatter_desc(0, 1).wait()


def kernel(input, weight):
    idx = jnp.asarray(input, jnp.int32).reshape(_B)
    out = _embed_sc(idx, weight)
    return out[:, :_D].reshape(_ROWS, _SEQ, _D)
